# baseline (speedup 1.0000x reference)
import functools

import jax
import jax.numpy as jnp
from jax import lax
from jax.experimental import pallas as pl
from jax.experimental.pallas import tpu as pltpu
from jax.experimental.pallas import tpu_sc as plsc

_TC_B = 24
_PIX = 512 * 512
_NW = 32
_STRIPE = _PIX // _NW
_CHUNK = 4096
_NCHUNK = _STRIPE // _CHUNK


def _tc_kernel(msk_ref, img_ref, lbl_ref, out_ref, acc_ref):
    i = pl.program_id(0)

    @pl.when(i == 0)
    def _init():
        acc_ref[0] = 0.0
        acc_ref[1] = 0.0

    d = img_ref[...] - lbl_ref[...]
    d2s = jnp.sum(d * d, axis=1)
    mf = msk_ref[...].astype(jnp.float32)
    acc_ref[0] += jnp.sum(d2s * mf)
    acc_ref[1] += jnp.sum(mf)

    @pl.when(i == pl.num_programs(0) - 1)
    def _fin():
        out_ref[0] = acc_ref[0]
        out_ref[1] = acc_ref[1]


def _tc_partial(image, label, mask_location):
    B, C, H, W = image.shape
    return pl.pallas_call(
        _tc_kernel,
        grid=(_TC_B,),
        in_specs=[
            pl.BlockSpec((1, H, W), lambda i: (i, 0, 0)),
            pl.BlockSpec((1, C, H, W), lambda i: (i, 0, 0, 0)),
            pl.BlockSpec((1, C, H, W), lambda i: (i, 0, 0, 0)),
        ],
        out_specs=pl.BlockSpec(memory_space=pltpu.SMEM),
        out_shape=jax.ShapeDtypeStruct((2,), jnp.float32),
        scratch_shapes=[pltpu.SMEM((2,), jnp.float32)],
    )(mask_location, image, label)


def _sc_body(img_hbm, lbl_hbm, msk_hbm, out_hbm,
             i00, i01, i02, i10, i11, i12,
             l00, l01, l02, l10, l11, l12,
             m0, m1, outbuf, sem0, sem1):
    wid = lax.axis_index("s") * 2 + lax.axis_index("c")
    nb = msk_hbm.shape[0] // _PIX
    imgbuf = ((i00, i01, i02), (i10, i11, i12))
    lblbuf = ((l00, l01, l02), (l10, l11, l12))
    mskbuf = (m0, m1)
    sems = (sem0, sem1)

    def issue(it, par):
        b, ck = divmod(it, _NCHUNK)
        pix0 = wid * _STRIPE + ck * _CHUNK
        handles = []
        for c in range(3):
            off = ((_TC_B + b) * 3 + c) * _PIX + pix0
            handles.append(pltpu.async_copy(
                img_hbm.at[pl.ds(off, _CHUNK)], imgbuf[par][c], sems[par]))
            handles.append(pltpu.async_copy(
                lbl_hbm.at[pl.ds(off, _CHUNK)], lblbuf[par][c], sems[par]))
        moff = b * _PIX + pix0
        handles.append(pltpu.async_copy(
            msk_hbm.at[pl.ds(moff, _CHUNK)], mskbuf[par], sems[par]))
        return handles

    acc = jnp.zeros((16,), jnp.float32)
    cnt = jnp.zeros((16,), jnp.float32)
    total = nb * _NCHUNK
    pending = {0: issue(0, 0)}
    for it in range(total):
        par = it % 2
        if it + 1 < total:
            pending[it + 1] = issue(it + 1, (it + 1) % 2)
        for h in pending.pop(it):
            h.wait()
        ib, lb, mb = imgbuf[par], lblbuf[par], mskbuf[par]

        def body(i, carry):
            a, cn = carry
            s = pl.ds(i * 16, 16)
            d0 = ib[0][s] - lb[0][s]
            d1 = ib[1][s] - lb[1][s]
            d2 = ib[2][s] - lb[2][s]
            mf = mb[s]
            a = a + (d0 * d0 + d1 * d1 + d2 * d2) * mf
            cn = cn + mf
            return a, cn

        acc, cnt = lax.fori_loop(0, _CHUNK // 16, body, (acc, cnt))

    outbuf[0, :] = acc
    outbuf[1, :] = cnt
    pltpu.sync_copy(outbuf, out_hbm.at[wid])


@functools.cache
def _sc_partial():
    return pl.kernel(
        _sc_body,
        mesh=plsc.VectorSubcoreMesh(
            core_axis_name="c", subcore_axis_name="s",
            num_cores=2, num_subcores=16),
        out_type=jax.ShapeDtypeStruct((_NW, 2, 16), jnp.float32),
        scratch_types=(
            [pltpu.VMEM((_CHUNK,), jnp.float32)] * 12
            + [pltpu.VMEM((_CHUNK,), jnp.float32)] * 2
            + [pltpu.VMEM((2, 16), jnp.float32),
               pltpu.SemaphoreType.DMA,
               pltpu.SemaphoreType.DMA]
        ),
    )


def kernel(image, label, mask_location):
    tc = _tc_partial(image, label, mask_location)
    img_flat = image.reshape(-1)
    lbl_flat = label.reshape(-1)
    mskf = mask_location[_TC_B:].astype(jnp.float32).reshape(-1)
    sc = _sc_partial()(img_flat, lbl_flat, mskf)
    num = tc[0] + jnp.sum(sc[:, 0, :])
    cnt = tc[1] + jnp.sum(sc[:, 1, :])
    return num / (3.0 * cnt)


# speedup vs baseline: 2.3556x; 2.3556x over previous
import functools

import jax
import jax.numpy as jnp
from jax import lax
from jax.experimental import pallas as pl
from jax.experimental.pallas import tpu as pltpu
from jax.experimental.pallas import tpu_sc as plsc

_TC_B = 24
_H = 512
_W = 512
_NW = 32
_STRIPE_ROWS = _H // _NW
_CHUNK_ROWS = 8
_NCHUNK = _STRIPE_ROWS // _CHUNK_ROWS
_VECS = _CHUNK_ROWS * _W // 16


def _tc_kernel(msk_ref, img_ref, lbl_ref, out_ref, acc_ref):
    i = pl.program_id(0)

    @pl.when(i == 0)
    def _init():
        acc_ref[0] = 0.0
        acc_ref[1] = 0.0

    d = img_ref[...] - lbl_ref[...]
    d2s = jnp.sum(d * d, axis=1)
    mf = msk_ref[...].astype(jnp.float32)
    acc_ref[0] += jnp.sum(d2s * mf)
    acc_ref[1] += jnp.sum(mf)

    @pl.when(i == pl.num_programs(0) - 1)
    def _fin():
        out_ref[0] = acc_ref[0]
        out_ref[1] = acc_ref[1]


def _tc_partial(image, label, mask_location):
    B, C, H, W = image.shape
    return pl.pallas_call(
        _tc_kernel,
        grid=(_TC_B,),
        in_specs=[
            pl.BlockSpec((1, H, W), lambda i: (i, 0, 0)),
            pl.BlockSpec((1, C, H, W), lambda i: (i, 0, 0, 0)),
            pl.BlockSpec((1, C, H, W), lambda i: (i, 0, 0, 0)),
        ],
        out_specs=pl.BlockSpec(memory_space=pltpu.SMEM),
        out_shape=jax.ShapeDtypeStruct((2,), jnp.float32),
        scratch_shapes=[pltpu.SMEM((2,), jnp.float32)],
    )(mask_location, image, label)


def _sc_body(img_hbm, lbl_hbm, msk_hbm, out_hbm,
             i00, i01, i02, i10, i11, i12,
             l00, l01, l02, l10, l11, l12,
             m0, m1, outbuf, sem0, sem1):
    wid = lax.axis_index("s") * 2 + lax.axis_index("c")
    nb = msk_hbm.shape[0]
    imgbuf = ((i00, i01, i02), (i10, i11, i12))
    lblbuf = ((l00, l01, l02), (l10, l11, l12))
    mskbuf = (m0, m1)
    sems = (sem0, sem1)

    def issue(it, par):
        b, ck = divmod(it, _NCHUNK)
        r0 = wid * _STRIPE_ROWS + ck * _CHUNK_ROWS
        handles = []
        for c in range(3):
            p = (_TC_B + b) * 3 + c
            handles.append(pltpu.async_copy(
                img_hbm.at[p, pl.ds(r0, _CHUNK_ROWS), :],
                imgbuf[par][c], sems[par]))
            handles.append(pltpu.async_copy(
                lbl_hbm.at[p, pl.ds(r0, _CHUNK_ROWS), :],
                lblbuf[par][c], sems[par]))
        handles.append(pltpu.async_copy(
            msk_hbm.at[b, pl.ds(r0, _CHUNK_ROWS), :],
            mskbuf[par], sems[par]))
        return handles

    acc = jnp.zeros((16,), jnp.float32)
    cnt = jnp.zeros((16,), jnp.float32)
    total = nb * _NCHUNK
    pending = {0: issue(0, 0)}
    for it in range(total):
        par = it % 2
        if it + 1 < total:
            pending[it + 1] = issue(it + 1, (it + 1) % 2)
        for h in pending.pop(it):
            h.wait()
        ib, lb, mb = imgbuf[par], lblbuf[par], mskbuf[par]

        def body(i, carry):
            a, cn = carry
            r = i >> 5
            s = pl.ds((i & 31) * 16, 16)
            d0 = ib[0][r, s] - lb[0][r, s]
            d1 = ib[1][r, s] - lb[1][r, s]
            d2 = ib[2][r, s] - lb[2][r, s]
            mf = mb[r, s]
            a = a + (d0 * d0 + d1 * d1 + d2 * d2) * mf
            cn = cn + mf
            return a, cn

        acc, cnt = lax.fori_loop(0, _VECS, body, (acc, cnt))

    outbuf[0, :] = acc
    outbuf[1, :] = cnt
    pltpu.sync_copy(outbuf, out_hbm.at[wid])


@functools.cache
def _sc_partial():
    buf = pltpu.VMEM((_CHUNK_ROWS, _W), jnp.float32)
    return pl.kernel(
        _sc_body,
        mesh=plsc.VectorSubcoreMesh(
            core_axis_name="c", subcore_axis_name="s",
            num_cores=2, num_subcores=16),
        out_type=jax.ShapeDtypeStruct((_NW, 2, 16), jnp.float32),
        scratch_types=(
            [buf] * 14
            + [pltpu.VMEM((2, 16), jnp.float32),
               pltpu.SemaphoreType.DMA,
               pltpu.SemaphoreType.DMA]
        ),
    )


def kernel(image, label, mask_location):
    B, C, H, W = image.shape
    tc = _tc_partial(image, label, mask_location)
    img_pl = image.reshape(B * C, H, W)
    lbl_pl = label.reshape(B * C, H, W)
    mskf = mask_location[_TC_B:].astype(jnp.float32)
    sc = _sc_partial()(img_pl, lbl_pl, mskf)
    num = tc[0] + jnp.sum(sc[:, 0, :])
    cnt = tc[1] + jnp.sum(sc[:, 1, :])
    return num / (3.0 * cnt)


# speedup vs baseline: 2.3592x; 1.0015x over previous
import functools

import jax
import jax.numpy as jnp
from jax import lax
from jax.experimental import pallas as pl
from jax.experimental.pallas import tpu as pltpu
from jax.experimental.pallas import tpu_sc as plsc

_TC_B = 24
_H = 512
_W = 512
_NW = 32
_STRIPE_ROWS = _H // _NW
_CHUNK_ROWS = 8
_NCHUNK = _STRIPE_ROWS // _CHUNK_ROWS
_VECS = _CHUNK_ROWS * _W // 16


def _tc_kernel(msk_ref, img_ref, lbl_ref, out_ref, acc_ref):
    i = pl.program_id(0)

    @pl.when(i == 0)
    def _init():
        acc_ref[0] = 0.0
        acc_ref[1] = 0.0

    d = img_ref[...] - lbl_ref[...]
    d2s = jnp.sum(d * d, axis=1)
    mf = msk_ref[...].astype(jnp.float32)
    acc_ref[0] += jnp.sum(d2s * mf)
    acc_ref[1] += jnp.sum(mf)

    @pl.when(i == pl.num_programs(0) - 1)
    def _fin():
        out_ref[0] = acc_ref[0]
        out_ref[1] = acc_ref[1]


def _tc_partial(image, label, mask_location):
    B, C, H, W = image.shape
    return pl.pallas_call(
        _tc_kernel,
        grid=(_TC_B,),
        in_specs=[
            pl.BlockSpec((1, H, W), lambda i: (i, 0, 0)),
            pl.BlockSpec((1, C, H, W), lambda i: (i, 0, 0, 0)),
            pl.BlockSpec((1, C, H, W), lambda i: (i, 0, 0, 0)),
        ],
        out_specs=pl.BlockSpec(memory_space=pltpu.SMEM),
        out_shape=jax.ShapeDtypeStruct((2,), jnp.float32),
        scratch_shapes=[pltpu.SMEM((2,), jnp.float32)],
    )(mask_location, image, label)


def _sc_body(img_hbm, lbl_hbm, msk_hbm, out_hbm,
             i00, i01, i02, i10, i11, i12,
             l00, l01, l02, l10, l11, l12,
             m0, m1, outbuf, sem0, sem1):
    wid = lax.axis_index("s") * 2 + lax.axis_index("c")
    nb = msk_hbm.shape[0]
    imgbuf = ((i00, i01, i02), (i10, i11, i12))
    lblbuf = ((l00, l01, l02), (l10, l11, l12))
    mskbuf = (m0, m1)
    sems = (sem0, sem1)

    def issue(it, par):
        b, ck = divmod(it, _NCHUNK)
        r0 = wid * _STRIPE_ROWS + ck * _CHUNK_ROWS
        handles = []
        for c in range(3):
            handles.append(pltpu.async_copy(
                img_hbm.at[_TC_B + b, c, pl.ds(r0, _CHUNK_ROWS), :],
                imgbuf[par][c], sems[par]))
            handles.append(pltpu.async_copy(
                lbl_hbm.at[_TC_B + b, c, pl.ds(r0, _CHUNK_ROWS), :],
                lblbuf[par][c], sems[par]))
        handles.append(pltpu.async_copy(
            msk_hbm.at[b, pl.ds(r0, _CHUNK_ROWS), :],
            mskbuf[par], sems[par]))
        return handles

    acc = jnp.zeros((16,), jnp.float32)
    cnt = jnp.zeros((16,), jnp.float32)
    total = nb * _NCHUNK
    pending = {0: issue(0, 0)}
    for it in range(total):
        par = it % 2
        if it + 1 < total:
            pending[it + 1] = issue(it + 1, (it + 1) % 2)
        for h in pending.pop(it):
            h.wait()
        ib, lb, mb = imgbuf[par], lblbuf[par], mskbuf[par]

        def body(i, carry):
            a, cn = carry
            r = i >> 5
            s = pl.ds((i & 31) * 16, 16)
            d0 = ib[0][r, s] - lb[0][r, s]
            d1 = ib[1][r, s] - lb[1][r, s]
            d2 = ib[2][r, s] - lb[2][r, s]
            mf = mb[r, s]
            a = a + (d0 * d0 + d1 * d1 + d2 * d2) * mf
            cn = cn + mf
            return a, cn

        acc, cnt = lax.fori_loop(0, _VECS, body, (acc, cnt))

    outbuf[0, :] = acc
    outbuf[1, :] = cnt
    pltpu.sync_copy(outbuf, out_hbm.at[wid])


@functools.cache
def _sc_partial():
    buf = pltpu.VMEM((_CHUNK_ROWS, _W), jnp.float32)
    return pl.kernel(
        _sc_body,
        mesh=plsc.VectorSubcoreMesh(
            core_axis_name="c", subcore_axis_name="s",
            num_cores=2, num_subcores=16),
        out_type=jax.ShapeDtypeStruct((_NW, 2, 16), jnp.float32),
        scratch_types=(
            [buf] * 14
            + [pltpu.VMEM((2, 16), jnp.float32),
               pltpu.SemaphoreType.DMA,
               pltpu.SemaphoreType.DMA]
        ),
    )


def kernel(image, label, mask_location):
    B, C, H, W = image.shape
    tc = _tc_partial(image, label, mask_location)
    mskf = mask_location[_TC_B:].astype(jnp.float32)
    sc = _sc_partial()(image, label, mskf)
    num = tc[0] + jnp.sum(sc[:, 0, :])
    cnt = tc[1] + jnp.sum(sc[:, 1, :])
    return num / (3.0 * cnt)


# speedup vs baseline: 2.5849x; 1.0957x over previous
import functools

import jax
import jax.numpy as jnp
from jax import lax
from jax.experimental import pallas as pl
from jax.experimental.pallas import tpu as pltpu
from jax.experimental.pallas import tpu_sc as plsc

_TC_B = 22
_H = 512
_W = 512
_NW = 32
_STRIPE_ROWS = _H // _NW
_CHUNK_ROWS = 8
_NCHUNK = _STRIPE_ROWS // _CHUNK_ROWS
_VECS = _CHUNK_ROWS * _W // 16


def _tc_kernel(msk_ref, img_ref, lbl_ref, out_ref, acc_ref):
    i = pl.program_id(0)

    @pl.when(i == 0)
    def _init():
        acc_ref[0] = 0.0
        acc_ref[1] = 0.0

    d = img_ref[...] - lbl_ref[...]
    d2s = jnp.sum(d * d, axis=1)
    mf = msk_ref[...].astype(jnp.float32)
    acc_ref[0] += jnp.sum(d2s * mf)
    acc_ref[1] += jnp.sum(mf)

    @pl.when(i == pl.num_programs(0) - 1)
    def _fin():
        out_ref[0] = acc_ref[0]
        out_ref[1] = acc_ref[1]


def _tc_partial(image, label, mask_location):
    B, C, H, W = image.shape
    return pl.pallas_call(
        _tc_kernel,
        grid=(_TC_B,),
        in_specs=[
            pl.BlockSpec((1, H, W), lambda i: (i, 0, 0)),
            pl.BlockSpec((1, C, H, W), lambda i: (i, 0, 0, 0)),
            pl.BlockSpec((1, C, H, W), lambda i: (i, 0, 0, 0)),
        ],
        out_specs=pl.BlockSpec(memory_space=pltpu.SMEM),
        out_shape=jax.ShapeDtypeStruct((2,), jnp.float32),
        scratch_shapes=[pltpu.SMEM((2,), jnp.float32)],
    )(mask_location, image, label)


def _sc_body(img_hbm, lbl_hbm, msk_hbm, out_hbm,
             i00, i01, i02, i10, i11, i12,
             l00, l01, l02, l10, l11, l12,
             m0, m1, outbuf, sem0, sem1):
    wid = lax.axis_index("s") * 2 + lax.axis_index("c")
    nb = msk_hbm.shape[0]
    imgbuf = ((i00, i01, i02), (i10, i11, i12))
    lblbuf = ((l00, l01, l02), (l10, l11, l12))
    mskbuf = (m0, m1)
    sems = (sem0, sem1)

    def issue(it, par):
        b, ck = divmod(it, _NCHUNK)
        r0 = wid * _STRIPE_ROWS + ck * _CHUNK_ROWS
        handles = []
        for c in range(3):
            handles.append(pltpu.async_copy(
                img_hbm.at[_TC_B + b, c, pl.ds(r0, _CHUNK_ROWS), :],
                imgbuf[par][c], sems[par]))
            handles.append(pltpu.async_copy(
                lbl_hbm.at[_TC_B + b, c, pl.ds(r0, _CHUNK_ROWS), :],
                lblbuf[par][c], sems[par]))
        handles.append(pltpu.async_copy(
            msk_hbm.at[b, pl.ds(r0, _CHUNK_ROWS), :],
            mskbuf[par], sems[par]))
        return handles

    acc = jnp.zeros((16,), jnp.float32)
    cnt = jnp.zeros((16,), jnp.float32)
    total = nb * _NCHUNK
    pending = {0: issue(0, 0)}
    for it in range(total):
        par = it % 2
        if it + 1 < total:
            pending[it + 1] = issue(it + 1, (it + 1) % 2)
        for h in pending.pop(it):
            h.wait()
        ib, lb, mb = imgbuf[par], lblbuf[par], mskbuf[par]

        def body(i, carry):
            a, cn = carry
            r = i >> 5
            s = pl.ds((i & 31) * 16, 16)
            d0 = ib[0][r, s] - lb[0][r, s]
            d1 = ib[1][r, s] - lb[1][r, s]
            d2 = ib[2][r, s] - lb[2][r, s]
            mf = mb[r, s]
            a = a + (d0 * d0 + d1 * d1 + d2 * d2) * mf
            cn = cn + mf
            return a, cn

        acc, cnt = lax.fori_loop(0, _VECS, body, (acc, cnt))

    outbuf[0, :] = acc
    outbuf[1, :] = cnt
    pltpu.sync_copy(outbuf, out_hbm.at[wid])


@functools.cache
def _sc_partial():
    buf = pltpu.VMEM((_CHUNK_ROWS, _W), jnp.float32)
    return pl.kernel(
        _sc_body,
        mesh=plsc.VectorSubcoreMesh(
            core_axis_name="c", subcore_axis_name="s",
            num_cores=2, num_subcores=16),
        out_type=jax.ShapeDtypeStruct((_NW, 2, 16), jnp.float32),
        scratch_types=(
            [buf] * 14
            + [pltpu.VMEM((2, 16), jnp.float32),
               pltpu.SemaphoreType.DMA,
               pltpu.SemaphoreType.DMA]
        ),
    )


def kernel(image, label, mask_location):
    B, C, H, W = image.shape
    msk8 = mask_location.view(jnp.int8)
    tc = _tc_partial(image, label, msk8)
    mskf = mask_location[_TC_B:].astype(jnp.float32)
    sc = _sc_partial()(image, label, mskf)
    num = tc[0] + jnp.sum(sc[:, 0, :])
    cnt = tc[1] + jnp.sum(sc[:, 1, :])
    return num / (3.0 * cnt)
